# baseline (speedup 1.0000x reference)
import numpy as np
import jax
import jax.numpy as jnp
from jax.experimental import pallas as pl

B = 128
S = 256
DIM = 768
EMB = 256
K = 8192
EPS = 1e-12
G = 8
NIMG = 2 * B
KT = 1024


def _np_consts():
    sel1 = np.zeros((9 * 64, 256), np.float32)
    for dh in range(3):
        for dw in range(3):
            t = dh * 3 + dw
            for i in range(8):
                for j in range(8):
                    h, w = 2 * i + dh - 1, 2 * j + dw - 1
                    if 0 <= h < 16 and 0 <= w < 16:
                        sel1[t * 64 + i * 8 + j, h * 16 + w] = 1.0
    sel2 = np.zeros((9 * 16 * G, 64 * G), np.float32)
    for dh in range(3):
        for dw in range(3):
            t = dh * 3 + dw
            for g in range(G):
                for i in range(4):
                    for j in range(4):
                        h, w = 2 * i + dh - 1, 2 * j + dw - 1
                        if 0 <= h < 8 and 0 <= w < 8:
                            sel2[t * 16 * G + g * 16 + i * 4 + j,
                                 g * 64 + h * 8 + w] = 1.0
    sel3 = np.zeros((9 * 4 * G, 16 * G), np.float32)
    for dh in range(3):
        for dw in range(3):
            t = dh * 3 + dw
            for g in range(G):
                for i in range(2):
                    for j in range(2):
                        h, w = i + dh, j + dw
                        sel3[t * 4 * G + g * 4 + i * 2 + j,
                             g * 16 + h * 4 + w] = 1.0
    gcat = np.zeros((4 * EMB, 4 * EMB), np.float32)
    for i in range(4):
        for j in range(EMB):
            p, c = j // 64, 4 * (j % 64) + i
            gcat[p * EMB + c, i * EMB + j] = 1.0
    return sel1, sel2, sel3, gcat


_SEL1, _SEL2, _SEL3, _GCAT = _np_consts()


def _shift(v, dh, dw):
    if dh == 0 and dw == 0:
        return v
    s = -4 * dh - dw
    rows = v.shape[0]
    r = jax.lax.broadcasted_iota(jnp.int32, (rows, 1), 0)
    keep = jnp.ones((rows, 1), jnp.bool_)
    if dh:
        keep = jnp.logical_and(keep, (r % 16) >= 4)
    if dw:
        keep = jnp.logical_and(keep, (r % 4) != 0)
    shifted = jnp.concatenate([jnp.zeros((s, v.shape[1]), v.dtype), v[:-s]],
                              axis=0)
    return jnp.where(keep, shifted, 0.0)


def _encoder_body(*refs):
    f32 = jnp.float32
    xph = refs[:16]
    (win_ref, bin_ref, w1_ref, b1_ref, w2_ref, b2_ref, w3_ref, b3_ref,
     z_ref) = refs[16:]
    win = win_ref[...]
    bin_ = bin_ref[...]
    y = {}
    for h2 in range(4):
        for w2 in range(4):
            xp = xph[h2 * 4 + w2][...].reshape(G * 16, DIM)
            y[(h2, w2)] = jnp.dot(xp, win, preferred_element_type=f32) + bin_
    a1 = {}
    b1 = b1_ref[...]
    for qh in range(2):
        for qw in range(2):
            acc = None
            for dh in range(3):
                for dw in range(3):
                    eh, ew = 2 * qh + dh - 1, 2 * qw + dw - 1
                    v = _shift(y[(eh % 4, ew % 4)],
                               -1 if eh < 0 else 0, -1 if ew < 0 else 0)
                    p = jnp.dot(v, w1_ref[dh * 3 + dw],
                                preferred_element_type=f32)
                    acc = p if acc is None else acc + p
            a1[(qh, qw)] = jax.nn.relu(acc + b1)
    acc = None
    for dh in range(3):
        for dw in range(3):
            v = _shift(a1[((dh - 1) % 2, (dw - 1) % 2)],
                       -1 if dh == 0 else 0, -1 if dw == 0 else 0)
            p = jnp.dot(v, w2_ref[dh * 3 + dw], preferred_element_type=f32)
            acc = p if acc is None else acc + p
    a2 = jax.nn.relu(acc + b2_ref[...])
    a2g = a2.reshape(G, 4, 4, EMB)
    acc = None
    for dh in range(3):
        for dw in range(3):
            v = a2g[:, dh:dh + 2, dw:dw + 2, :].reshape(G * 4, EMB)
            p = jnp.dot(v, w3_ref[dh * 3 + dw], preferred_element_type=f32)
            acc = p if acc is None else acc + p
    z_ref[...] = acc + b3_ref[...]


def _vq_body(zf_ref, zl_ref, cbt_ref, cb_ref, noise_ref, q_ref, idx_ref):
    f32 = jnp.float32
    x = zl_ref[...] - zf_ref[...]
    xn2 = jnp.sum(x * x, axis=1, keepdims=True)
    best = jnp.full((4 * B, 1), jnp.inf, f32)
    bidx = jnp.zeros((4 * B, 1), jnp.int32)
    lane = jax.lax.broadcasted_iota(jnp.int32, (4 * B, KT), 1)
    for t in range(K // KT):
        cbt = cbt_ref[:, t * KT:(t + 1) * KT]
        cn2 = jnp.sum(cbt * cbt, axis=0, keepdims=True)
        s = xn2 - 2.0 * jnp.dot(x, cbt, preferred_element_type=f32) + cn2
        m = jnp.min(s, axis=1, keepdims=True)
        li = jnp.min(jnp.where(s <= m, lane, K), axis=1, keepdims=True) + t * KT
        upd = m < best
        bidx = jnp.where(upd, li, bidx)
        best = jnp.where(upd, m, best)
    hq = jnp.zeros((4 * B, EMB), f32)
    for t in range(K // KT):
        oh = (bidx == (lane + t * KT)).astype(f32)
        hq = hq + jnp.dot(oh, cb_ref[t * KT:(t + 1) * KT, :],
                          preferred_element_type=f32)
    r = x - hq
    nres = jnp.sqrt(jnp.sum(r * r, axis=1, keepdims=True))
    noise = noise_ref[...]
    nrand = jnp.sqrt(jnp.sum(noise * noise, axis=1, keepdims=True))
    q_ref[...] = x + (nres / nrand + EPS) * noise
    idx_ref[...] = bidx


def _out_body(qf_ref, idx_ref, gcat_ref, wout_ref, bout_ref, out_ref, ppl_ref):
    f32 = jnp.float32
    qd = jnp.dot(qf_ref[...], gcat_ref[...], preferred_element_type=f32)
    outs = []
    for i in range(4):
        o = jnp.dot(qd[:, i * EMB:(i + 1) * EMB], wout_ref[...],
                    preferred_element_type=f32) + bout_ref[...]
        outs.append(o[None])
    out_ref[...] = jnp.concatenate(outs, axis=0)
    idx = idx_ref[...]
    lane = jax.lax.broadcasted_iota(jnp.int32, (4 * B, KT), 1)
    tot = jnp.float32(0.0)
    for t in range(K // KT):
        cnt = jnp.sum((idx == (lane + t * KT)).astype(f32), axis=0,
                      keepdims=True)
        p = cnt / jnp.float32(4 * B)
        tot = tot + jnp.sum(p * jnp.log(p + 1e-10))
    ppl_ref[...] = jnp.full((1, 1), 1.0, f32) * jnp.exp(-tot)


def kernel(input_data_first, input_data_last, codebooks, W_in, b_in,
           conv1_w, conv1_b, conv2_w, conv2_b, conv3_w, conv3_b,
           W_out, b_out, noise):
    f32 = jnp.float32
    gcat = jnp.asarray(_GCAT)

    w1 = conv1_w.reshape(9, EMB, EMB)
    w2 = conv2_w.reshape(9, EMB, EMB)
    w3 = conv3_w.reshape(9, EMB, EMB)

    full = lambda shape: pl.BlockSpec(shape, lambda g: (0,) * len(shape))

    def phase_spec(h2, w2):
        return pl.BlockSpec((G, 4, 1, 4, DIM),
                            lambda g, h2=h2, w2=w2: (g, 0, h2, 0, w2))

    encode = pl.pallas_call(
        _encoder_body,
        grid=(B // G,),
        in_specs=[phase_spec(h2, w2) for h2 in range(4) for w2 in range(4)]
        + [
            full((DIM, EMB)),
            full((1, EMB)),
            full((9, EMB, EMB)),
            full((1, EMB)),
            full((9, EMB, EMB)),
            full((1, EMB)),
            full((9, EMB, EMB)),
            full((1, EMB)),
        ],
        out_specs=pl.BlockSpec((G * 4, EMB), lambda g: (g, 0)),
        out_shape=jax.ShapeDtypeStruct((B * 4, EMB), f32),
    )
    wargs = (W_in, b_in.reshape(1, EMB),
             w1, conv1_b.reshape(1, EMB), w2, conv2_b.reshape(1, EMB),
             w3, conv3_b.reshape(1, EMB))

    def run_encode(inp):
        inp6 = inp.reshape(B, 4, 4, 4, 4 * DIM)
        return encode(*([inp6] * 16), *wargs)

    zf = run_encode(input_data_first)
    zl = run_encode(input_data_last)

    q, idx = pl.pallas_call(
        _vq_body,
        out_shape=(jax.ShapeDtypeStruct((4 * B, EMB), f32),
                   jax.ShapeDtypeStruct((4 * B, 1), jnp.int32)),
    )(zf, zl, codebooks.T, codebooks, noise)

    qf = q.reshape(B, 4 * EMB)
    out4, ppl = pl.pallas_call(
        _out_body,
        out_shape=(jax.ShapeDtypeStruct((4, B, DIM), f32),
                   jax.ShapeDtypeStruct((1, 1), f32)),
    )(qf, idx, gcat, W_out, b_out.reshape(1, DIM))

    out = jnp.transpose(out4, (1, 0, 2))
    return out, ppl.reshape(()), idx.reshape(4 * B)


# speedup vs baseline: 1.8134x; 1.8134x over previous
import numpy as np
import jax
import jax.numpy as jnp
from jax.experimental import pallas as pl

B = 128
S = 256
DIM = 768
EMB = 256
K = 8192
EPS = 1e-12
G = 8
NIMG = 2 * B
KT = 1024


def _np_consts():
    sel1 = np.zeros((9 * 64, 256), np.float32)
    for dh in range(3):
        for dw in range(3):
            t = dh * 3 + dw
            for i in range(8):
                for j in range(8):
                    h, w = 2 * i + dh - 1, 2 * j + dw - 1
                    if 0 <= h < 16 and 0 <= w < 16:
                        sel1[t * 64 + i * 8 + j, h * 16 + w] = 1.0
    sel2 = np.zeros((9 * 16 * G, 64 * G), np.float32)
    for dh in range(3):
        for dw in range(3):
            t = dh * 3 + dw
            for g in range(G):
                for i in range(4):
                    for j in range(4):
                        h, w = 2 * i + dh - 1, 2 * j + dw - 1
                        if 0 <= h < 8 and 0 <= w < 8:
                            sel2[t * 16 * G + g * 16 + i * 4 + j,
                                 g * 64 + h * 8 + w] = 1.0
    sel3 = np.zeros((9 * 4 * G, 16 * G), np.float32)
    for dh in range(3):
        for dw in range(3):
            t = dh * 3 + dw
            for g in range(G):
                for i in range(2):
                    for j in range(2):
                        h, w = i + dh, j + dw
                        sel3[t * 4 * G + g * 4 + i * 2 + j,
                             g * 16 + h * 4 + w] = 1.0
    gcat = np.zeros((4 * EMB, 4 * EMB), np.float32)
    for i in range(4):
        for j in range(EMB):
            p, c = j // 64, 4 * (j % 64) + i
            gcat[p * EMB + c, i * EMB + j] = 1.0
    return sel1, sel2, sel3, gcat


_SEL1, _SEL2, _SEL3, _GCAT = _np_consts()


def _shift(v, dh, dw):
    if dh == 0 and dw == 0:
        return v
    s = -4 * dh - dw
    rows = v.shape[0]
    r = jax.lax.broadcasted_iota(jnp.int32, (rows, 1), 0)
    keep = jnp.ones((rows, 1), jnp.bool_)
    if dh:
        keep = jnp.logical_and(keep, (r % 16) >= 4)
    if dw:
        keep = jnp.logical_and(keep, (r % 4) != 0)
    shifted = jnp.concatenate([jnp.zeros((s, v.shape[1]), v.dtype), v[:-s]],
                              axis=0)
    return jnp.where(keep, shifted, 0.0)


def _encoder_body(x_ref, win_ref, bin_ref, w1_ref, b1_ref, w2_ref, b2_ref,
                  w3_ref, b3_ref, z_ref):
    f32 = jnp.float32
    X = x_ref[...].reshape(G * S, DIM)
    Y = jnp.dot(X, win_ref[...], preferred_element_type=f32) + bin_ref[...]
    Y6 = Y.reshape(G, 4, 4, 4, 4, EMB)
    y = {}
    for h2 in range(4):
        for w2 in range(4):
            y[(h2, w2)] = Y6[:, :, h2, :, w2, :].reshape(G * 16, EMB)
    a1 = {}
    b1 = b1_ref[...]
    for qh in range(2):
        for qw in range(2):
            acc = None
            for dh in range(3):
                for dw in range(3):
                    eh, ew = 2 * qh + dh - 1, 2 * qw + dw - 1
                    v = _shift(y[(eh % 4, ew % 4)],
                               -1 if eh < 0 else 0, -1 if ew < 0 else 0)
                    p = jnp.dot(v, w1_ref[dh * 3 + dw],
                                preferred_element_type=f32)
                    acc = p if acc is None else acc + p
            a1[(qh, qw)] = jax.nn.relu(acc + b1)
    acc = None
    for dh in range(3):
        for dw in range(3):
            v = _shift(a1[((dh - 1) % 2, (dw - 1) % 2)],
                       -1 if dh == 0 else 0, -1 if dw == 0 else 0)
            p = jnp.dot(v, w2_ref[dh * 3 + dw], preferred_element_type=f32)
            acc = p if acc is None else acc + p
    a2 = jax.nn.relu(acc + b2_ref[...])
    a2g = a2.reshape(G, 4, 4, EMB)
    acc = None
    for dh in range(3):
        for dw in range(3):
            v = a2g[:, dh:dh + 2, dw:dw + 2, :].reshape(G * 4, EMB)
            p = jnp.dot(v, w3_ref[dh * 3 + dw], preferred_element_type=f32)
            acc = p if acc is None else acc + p
    z_ref[...] = acc + b3_ref[...]


def _vq_body(zf_ref, zl_ref, cbt_ref, cb_ref, noise_ref, q_ref, idx_ref):
    f32 = jnp.float32
    x = zl_ref[...] - zf_ref[...]
    xn2 = jnp.sum(x * x, axis=1, keepdims=True)
    best = jnp.full((4 * B, 1), jnp.inf, f32)
    bidx = jnp.zeros((4 * B, 1), jnp.int32)
    lane = jax.lax.broadcasted_iota(jnp.int32, (4 * B, KT), 1)
    for t in range(K // KT):
        cbt = cbt_ref[:, t * KT:(t + 1) * KT]
        cn2 = jnp.sum(cbt * cbt, axis=0, keepdims=True)
        s = xn2 - 2.0 * jnp.dot(x, cbt, preferred_element_type=f32) + cn2
        m = jnp.min(s, axis=1, keepdims=True)
        li = jnp.min(jnp.where(s <= m, lane, K), axis=1, keepdims=True) + t * KT
        upd = m < best
        bidx = jnp.where(upd, li, bidx)
        best = jnp.where(upd, m, best)
    hq = jnp.zeros((4 * B, EMB), f32)
    for t in range(K // KT):
        oh = (bidx == (lane + t * KT)).astype(f32)
        hq = hq + jnp.dot(oh, cb_ref[t * KT:(t + 1) * KT, :],
                          preferred_element_type=f32)
    r = x - hq
    nres = jnp.sqrt(jnp.sum(r * r, axis=1, keepdims=True))
    noise = noise_ref[...]
    nrand = jnp.sqrt(jnp.sum(noise * noise, axis=1, keepdims=True))
    q_ref[...] = x + (nres / nrand + EPS) * noise
    idx_ref[...] = bidx


def _out_body(qf_ref, idx_ref, gcat_ref, wout_ref, bout_ref, out_ref, ppl_ref):
    f32 = jnp.float32
    qd = jnp.dot(qf_ref[...], gcat_ref[...], preferred_element_type=f32)
    outs = []
    for i in range(4):
        o = jnp.dot(qd[:, i * EMB:(i + 1) * EMB], wout_ref[...],
                    preferred_element_type=f32) + bout_ref[...]
        outs.append(o[None])
    out_ref[...] = jnp.concatenate(outs, axis=0)
    idx = idx_ref[...]
    lane = jax.lax.broadcasted_iota(jnp.int32, (4 * B, KT), 1)
    tot = jnp.float32(0.0)
    for t in range(K // KT):
        cnt = jnp.sum((idx == (lane + t * KT)).astype(f32), axis=0,
                      keepdims=True)
        p = cnt / jnp.float32(4 * B)
        tot = tot + jnp.sum(p * jnp.log(p + 1e-10))
    ppl_ref[...] = jnp.full((1, 1), 1.0, f32) * jnp.exp(-tot)


def kernel(input_data_first, input_data_last, codebooks, W_in, b_in,
           conv1_w, conv1_b, conv2_w, conv2_b, conv3_w, conv3_b,
           W_out, b_out, noise):
    f32 = jnp.float32
    gcat = jnp.asarray(_GCAT)

    w1 = conv1_w.reshape(9, EMB, EMB)
    w2 = conv2_w.reshape(9, EMB, EMB)
    w3 = conv3_w.reshape(9, EMB, EMB)

    full = lambda shape: pl.BlockSpec(shape, lambda g: (0,) * len(shape))

    encode = pl.pallas_call(
        _encoder_body,
        grid=(B // G,),
        in_specs=[
            pl.BlockSpec((G, S, DIM), lambda g: (g, 0, 0)),
            full((DIM, EMB)),
            full((1, EMB)),
            full((9, EMB, EMB)),
            full((1, EMB)),
            full((9, EMB, EMB)),
            full((1, EMB)),
            full((9, EMB, EMB)),
            full((1, EMB)),
        ],
        out_specs=pl.BlockSpec((G * 4, EMB), lambda g: (g, 0)),
        out_shape=jax.ShapeDtypeStruct((B * 4, EMB), f32),
    )
    wargs = (W_in, b_in.reshape(1, EMB),
             w1, conv1_b.reshape(1, EMB), w2, conv2_b.reshape(1, EMB),
             w3, conv3_b.reshape(1, EMB))

    zf = encode(input_data_first, *wargs)
    zl = encode(input_data_last, *wargs)

    q, idx = pl.pallas_call(
        _vq_body,
        out_shape=(jax.ShapeDtypeStruct((4 * B, EMB), f32),
                   jax.ShapeDtypeStruct((4 * B, 1), jnp.int32)),
    )(zf, zl, codebooks.T, codebooks, noise)

    qf = q.reshape(B, 4 * EMB)
    out4, ppl = pl.pallas_call(
        _out_body,
        out_shape=(jax.ShapeDtypeStruct((4, B, DIM), f32),
                   jax.ShapeDtypeStruct((1, 1), f32)),
    )(qf, idx, gcat, W_out, b_out.reshape(1, DIM))

    out = jnp.transpose(out4, (1, 0, 2))
    return out, ppl.reshape(()), idx.reshape(4 * B)
